# baseline (speedup 1.0000x reference)
import functools

import jax
import jax.numpy as jnp
from jax import lax
from jax.experimental import pallas as pl
from jax.experimental.pallas import tpu as pltpu
from jax.experimental.pallas import tpu_sc as plsc

N = 10000
E = 320000
D = 128
NG = 128

NC = 2
NS = 16
NW = NC * NS
EPT = E // NW
CH = 80
NCHUNK = EPT // CH
RPT = 624
ZR = 48
NROW = 4
NIDX = 8
GL = 2
IL = 4


def _sc_agg_body(h_hbm, ei_hbm, out_hbm,
                 src0, src1, src2, src3, src4, src5, src6, src7,
                 dst0, dst1, dst2, dst3, dst4, dst5, dst6, dst7,
                 rows0, rows1, rows2, rows3,
                 zbuf_v, agg_sh, isem, gsem, ssem):
    cid = lax.axis_index("c")
    sid = lax.axis_index("s")
    wid = cid * NS + sid
    srcs = (src0, src1, src2, src3, src4, src5, src6, src7)
    dsts = (dst0, dst1, dst2, dst3, dst4, dst5, dst6, dst7)
    rows = (rows0, rows1, rows2, rows3)
    ebase = wid * EPT

    def idx_load(i, q):
        pltpu.async_copy(ei_hbm.at[pl.ds(ebase + i * CH, CH)], srcs[q],
                         isem)
        pltpu.async_copy(ei_hbm.at[pl.ds(E + ebase + i * CH, CH)], dsts[q],
                         isem)

    def idx_wait(i, q):
        pltpu.make_async_copy(ei_hbm.at[pl.ds(ebase + i * CH, CH)],
                              srcs[q], isem).wait()
        pltpu.make_async_copy(ei_hbm.at[pl.ds(E + ebase + i * CH, CH)],
                              dsts[q], isem).wait()

    def scatter_drain():
        pltpu.make_async_copy(rows[0], agg_sh.at[dsts[0]], ssem).wait()

    for q in range(NIDX):
        idx_load(q, q)

    z16 = jnp.zeros((16,), jnp.float32)

    def zrow(i, _):
        def zcol(j, __):
            zbuf_v[i, pl.ds(j * 16, 16)] = z16
            return __
        return lax.fori_loop(0, D // 16, zcol, None)

    lax.fori_loop(0, ZR, zrow, None)
    for k in range(RPT // ZR):
        pltpu.sync_copy(zbuf_v, agg_sh.at[pl.ds(sid * RPT + k * ZR, ZR)])

    @pl.when(sid == NS - 1)
    def _zero_tail():
        pltpu.sync_copy(zbuf_v.at[pl.ds(0, 16)],
                        agg_sh.at[pl.ds(NS * RPT, 16)])

    for j in range(GL):
        idx_wait(j, j)
        pltpu.async_copy(h_hbm.at[srcs[j]], rows[j], gsem)
    plsc.subcore_barrier()

    def step(i, r, q, head):
        pltpu.make_async_copy(h_hbm.at[srcs[q]], rows[r], gsem).wait()
        pltpu.async_copy(rows[r], agg_sh.at[dsts[q]], ssem, add=True)
        if head is None or head >= 2:
            scatter_drain()
        if head is None or (NIDX <= head + IL < NCHUNK):
            idx_load(i + IL, (q + IL) % NIDX)
        if head is None or head + GL < NCHUNK:
            q2, r2 = (q + GL) % NIDX, (r + GL) % NROW
            idx_wait(i + GL, q2)
            pltpu.async_copy(h_hbm.at[srcs[q2]], rows[r2], gsem)

    for i in range(NIDX):
        step(i, i % NROW, i % NIDX, head=i)

    def outer(io, _):
        i = io * NIDX
        for b in range(NIDX):
            step(i + b, b % NROW, b % NIDX, head=None)
        return _

    steady_end = (NCHUNK // NIDX) * NIDX
    lax.fori_loop(1, steady_end // NIDX, outer, None)
    for i in range(steady_end, NCHUNK):
        step(i, i % NROW, i % NIDX, head=i)

    for _ in range(2):
        scatter_drain()
    plsc.subcore_barrier()

    pltpu.sync_copy(agg_sh.at[pl.ds(sid * RPT, RPT)],
                    out_hbm.at[pl.ds(cid * N + sid * RPT, RPT)])

    @pl.when(sid == NS - 1)
    def _write_tail():
        pltpu.sync_copy(agg_sh.at[pl.ds(NS * RPT, 16)],
                        out_hbm.at[pl.ds(cid * N + NS * RPT, 16)])


@jax.jit
def _sc_agg(h, ei):
    mesh = plsc.VectorSubcoreMesh(core_axis_name="c", subcore_axis_name="s",
                                  num_cores=NC, num_subcores=NS)
    return pl.kernel(
        _sc_agg_body,
        out_type=jax.ShapeDtypeStruct((NC * N, D), jnp.float32),
        mesh=mesh,
        scratch_types=(
            [pltpu.VMEM((CH,), jnp.int32)] * 16
            + [pltpu.VMEM((CH, D), jnp.float32)] * 4
            + [
                pltpu.VMEM((ZR, D), jnp.float32),
                pltpu.VMEM_SHARED((N, D), jnp.float32),
                pltpu.SemaphoreType.DMA,
                pltpu.SemaphoreType.DMA,
                pltpu.SemaphoreType.DMA,
            ]
        ),
    )(h, ei)


OUT_DIM = 64


BR = 1000
NB = N // BR


def _tc_dense_body(h_ref, agg_ref, W_ref, b_ref, g_ref, be_ref,
                   batch_ref, Wo_ref, bo_ref, sin_ref,
                   out_ref, score_ref, pre_sc, acc_sc, stat_sc, pool_sc):
    p = pl.program_id(0)
    i = pl.program_id(1)

    @pl.when(p == 0)
    def _compute():
        hs = h_ref[...] + agg_ref[0] + agg_ref[1]
        pre = jnp.dot(hs, W_ref[...], preferred_element_type=jnp.float32)
        pre = pre + b_ref[...][None, :]
        pre_sc[pl.ds(i * BR, BR), :] = pre
        psum = jnp.sum(pre, axis=0, keepdims=True)
        psq = jnp.sum(pre * pre, axis=0, keepdims=True)

        @pl.when(i == 0)
        def _init():
            acc_sc[0:1, :] = psum
            acc_sc[1:2, :] = psq

        @pl.when(i > 0)
        def _acc():
            acc_sc[0:1, :] += psum
            acc_sc[1:2, :] += psq

    @pl.when(p == 1)
    def _normalize():
        @pl.when(i == 0)
        def _stats():
            mu = acc_sc[0:1, :] * (1.0 / N)
            var = acc_sc[1:2, :] * (1.0 / N) - mu * mu
            inv = lax.rsqrt(var + 1e-5) * g_ref[...][None, :]
            stat_sc[0:1, :] = inv
            stat_sc[1:2, :] = be_ref[...][None, :] - mu * inv

        pre = pre_sc[pl.ds(i * BR, BR), :]
        h_out = jnp.maximum(pre * stat_sc[0:1, :] + stat_sc[1:2, :], 0.0)
        out_ref[...] = h_out
        seg = lax.broadcasted_iota(jnp.int32, (NG, BR), 0)
        onehot = (seg == batch_ref[0, 0, :][None, :]).astype(jnp.float32)
        pooled = jnp.dot(onehot, h_out, preferred_element_type=jnp.float32)

        @pl.when(i == 0)
        def _pool_init():
            pool_sc[...] = pooled

        @pl.when(i > 0)
        def _pool_acc():
            pool_sc[...] += pooled

        @pl.when(i == NB - 1)
        def _head():
            score_ref[...] = (sin_ref[...]
                              + jnp.dot(pool_sc[...], Wo_ref[...],
                                        preferred_element_type=jnp.float32)
                              + bo_ref[...][None, :])


@jax.jit
def _tc_dense(h, agg2, W, b, g, be, batch, Wo, bo, score_in):
    zero = lambda p, i: (0, 0)
    first = lambda p, i: (i * (1 - p), 0)
    return pl.pallas_call(
        _tc_dense_body,
        grid=(2, NB),
        in_specs=[
            pl.BlockSpec((BR, D), first),
            pl.BlockSpec((NC, BR, D), lambda p, i: (0, i * (1 - p), 0)),
            pl.BlockSpec((D, D), zero),
            pl.BlockSpec((D,), lambda p, i: (0,)),
            pl.BlockSpec((D,), lambda p, i: (0,)),
            pl.BlockSpec((D,), lambda p, i: (0,)),
            pl.BlockSpec((1, 1, BR), lambda p, i: (p * i, 0, 0)),
            pl.BlockSpec((D, OUT_DIM), zero),
            pl.BlockSpec((OUT_DIM,), lambda p, i: (0,)),
            pl.BlockSpec((NG, OUT_DIM), zero),
        ],
        out_specs=(pl.BlockSpec((BR, D), lambda p, i: (p * i, 0)),
                   pl.BlockSpec((NG, OUT_DIM), zero)),
        out_shape=(jax.ShapeDtypeStruct((N, D), jnp.float32),
                   jax.ShapeDtypeStruct((NG, OUT_DIM), jnp.float32)),
        scratch_shapes=[
            pltpu.VMEM((N, D), jnp.float32),
            pltpu.VMEM((8, D), jnp.float32),
            pltpu.VMEM((8, D), jnp.float32),
            pltpu.VMEM((NG, D), jnp.float32),
        ],
    )(h, agg2.reshape(NC, N, D), W, b, g, be,
      batch.reshape(NB, 1, BR), Wo, bo, score_in)


def kernel(x, edge_index, batch, W0, b0, g0, be0, W1, b1, g1, be1,
           W2, b2, g2, be2, Wo0, bo0, Wo1, bo1, Wo2, bo2):
    ei = edge_index.reshape(2 * E)
    score = jnp.zeros((NG, OUT_DIM), jnp.float32)

    agg = _sc_agg(x, ei)
    h0, score = _tc_dense(x, agg, W0, b0, g0, be0, batch, Wo0, bo0, score)
    agg = _sc_agg(h0, ei)
    h1, score = _tc_dense(h0, agg, W1, b1, g1, be1, batch, Wo1, bo1, score)
    agg = _sc_agg(h1, ei)
    h2, score = _tc_dense(h1, agg, W2, b2, g2, be2, batch, Wo2, bo2, score)

    return score


# speedup vs baseline: 1.0833x; 1.0833x over previous
import functools

import jax
import jax.numpy as jnp
from jax import lax
from jax.experimental import pallas as pl
from jax.experimental.pallas import tpu as pltpu
from jax.experimental.pallas import tpu_sc as plsc

N = 10000
E = 320000
D = 128
NG = 128

NC = 2
NS = 16
NW = NC * NS
EPT = E // NW
CH = 80
NCHUNK = EPT // CH
RPT = 624
ZR = 48
NROW = 4
NIDX = 8
GL = 2
IL = 4


def _sc_agg_body(h_hbm, ei_hbm, out_hbm,
                 src0, src1, src2, src3, src4, src5, src6, src7,
                 dst0, dst1, dst2, dst3, dst4, dst5, dst6, dst7,
                 rows0, rows1, rows2, rows3,
                 zbuf_v, agg_sh, isem, gsem, ssem):
    cid = lax.axis_index("c")
    sid = lax.axis_index("s")
    wid = cid * NS + sid
    srcs = (src0, src1, src2, src3, src4, src5, src6, src7)
    dsts = (dst0, dst1, dst2, dst3, dst4, dst5, dst6, dst7)
    rows = (rows0, rows1, rows2, rows3)
    ebase = wid * EPT

    def idx_load(i, q):
        pltpu.async_copy(ei_hbm.at[pl.ds(ebase + i * CH, CH)], srcs[q],
                         isem)
        pltpu.async_copy(ei_hbm.at[pl.ds(E + ebase + i * CH, CH)], dsts[q],
                         isem)

    def idx_wait(i, q):
        pltpu.make_async_copy(ei_hbm.at[pl.ds(ebase + i * CH, CH)],
                              srcs[q], isem).wait()
        pltpu.make_async_copy(ei_hbm.at[pl.ds(E + ebase + i * CH, CH)],
                              dsts[q], isem).wait()

    def scatter_drain():
        pltpu.make_async_copy(rows[0], agg_sh.at[dsts[0]], ssem).wait()

    for q in range(NIDX):
        idx_load(q, q)

    z16 = jnp.zeros((16,), jnp.float32)

    def zrow(i, _):
        def zcol(j, __):
            zbuf_v[i, pl.ds(j * 16, 16)] = z16
            return __
        return lax.fori_loop(0, D // 16, zcol, None)

    lax.fori_loop(0, ZR, zrow, None)
    for k in range(RPT // ZR):
        pltpu.sync_copy(zbuf_v, agg_sh.at[pl.ds(sid * RPT + k * ZR, ZR)])

    @pl.when(sid == NS - 1)
    def _zero_tail():
        pltpu.sync_copy(zbuf_v.at[pl.ds(0, 16)],
                        agg_sh.at[pl.ds(NS * RPT, 16)])

    for j in range(GL):
        idx_wait(j, j)
        pltpu.async_copy(h_hbm.at[srcs[j]], rows[j], gsem)
    plsc.subcore_barrier()

    def step(i, r, q, head):
        pltpu.make_async_copy(h_hbm.at[srcs[q]], rows[r], gsem).wait()
        pltpu.async_copy(rows[r], agg_sh.at[dsts[q]], ssem, add=True)
        if head is None or head >= 2:
            scatter_drain()
        if head is None or (NIDX <= head + IL < NCHUNK):
            idx_load(i + IL, (q + IL) % NIDX)
        if head is None or head + GL < NCHUNK:
            q2, r2 = (q + GL) % NIDX, (r + GL) % NROW
            idx_wait(i + GL, q2)
            pltpu.async_copy(h_hbm.at[srcs[q2]], rows[r2], gsem)

    for i in range(NIDX):
        step(i, i % NROW, i % NIDX, head=i)

    def outer(io, _):
        i = io * NIDX
        for b in range(NIDX):
            step(i + b, b % NROW, b % NIDX, head=None)
        return _

    steady_end = (NCHUNK // NIDX) * NIDX
    lax.fori_loop(1, steady_end // NIDX, outer, None)
    for i in range(steady_end, NCHUNK):
        step(i, i % NROW, i % NIDX, head=i)

    for _ in range(2):
        scatter_drain()
    plsc.subcore_barrier()

    pltpu.sync_copy(agg_sh.at[pl.ds(sid * RPT, RPT)],
                    out_hbm.at[pl.ds(cid * N + sid * RPT, RPT)])

    @pl.when(sid == NS - 1)
    def _write_tail():
        pltpu.sync_copy(agg_sh.at[pl.ds(NS * RPT, 16)],
                        out_hbm.at[pl.ds(cid * N + NS * RPT, 16)])


@jax.jit
def _sc_agg(h, ei):
    mesh = plsc.VectorSubcoreMesh(core_axis_name="c", subcore_axis_name="s",
                                  num_cores=NC, num_subcores=NS)
    return pl.kernel(
        _sc_agg_body,
        out_type=jax.ShapeDtypeStruct((NC * N, D), jnp.float32),
        mesh=mesh,
        scratch_types=(
            [pltpu.VMEM((CH,), jnp.int32)] * 16
            + [pltpu.VMEM((CH, D), jnp.float32)] * 4
            + [
                pltpu.VMEM((ZR, D), jnp.float32),
                pltpu.VMEM_SHARED((N, D), jnp.float32),
                pltpu.SemaphoreType.DMA,
                pltpu.SemaphoreType.DMA,
                pltpu.SemaphoreType.DMA,
            ]
        ),
    )(h, ei)


OUT_DIM = 64


BR = 2000
NB = N // BR


def _tc_dense_body(h_ref, agg_ref, W_ref, b_ref, g_ref, be_ref,
                   batch_ref, Wo_ref, bo_ref, sin_ref,
                   out_ref, score_ref, pre_sc, acc_sc, stat_sc, pool_sc):
    p = pl.program_id(0)
    i = pl.program_id(1)

    @pl.when(p == 0)
    def _compute():
        hs = h_ref[...] + agg_ref[0] + agg_ref[1]
        pre = jnp.dot(hs, W_ref[...], preferred_element_type=jnp.float32)
        pre = pre + b_ref[...][None, :]
        pre_sc[pl.ds(i * BR, BR), :] = pre
        psum = jnp.sum(pre, axis=0, keepdims=True)
        psq = jnp.sum(pre * pre, axis=0, keepdims=True)

        @pl.when(i == 0)
        def _init():
            acc_sc[0:1, :] = psum
            acc_sc[1:2, :] = psq

        @pl.when(i > 0)
        def _acc():
            acc_sc[0:1, :] += psum
            acc_sc[1:2, :] += psq

    @pl.when(p == 1)
    def _normalize():
        @pl.when(i == 0)
        def _stats():
            mu = acc_sc[0:1, :] * (1.0 / N)
            var = acc_sc[1:2, :] * (1.0 / N) - mu * mu
            inv = lax.rsqrt(var + 1e-5) * g_ref[...][None, :]
            stat_sc[0:1, :] = inv
            stat_sc[1:2, :] = be_ref[...][None, :] - mu * inv

        pre = pre_sc[pl.ds(i * BR, BR), :]
        h_out = jnp.maximum(pre * stat_sc[0:1, :] + stat_sc[1:2, :], 0.0)
        out_ref[...] = h_out
        seg = lax.broadcasted_iota(jnp.int32, (NG, BR), 0)
        onehot = (seg == batch_ref[0, 0, :][None, :]).astype(jnp.float32)
        pooled = jnp.dot(onehot, h_out, preferred_element_type=jnp.float32)

        @pl.when(i == 0)
        def _pool_init():
            pool_sc[...] = pooled

        @pl.when(i > 0)
        def _pool_acc():
            pool_sc[...] += pooled

        @pl.when(i == NB - 1)
        def _head():
            score_ref[...] = (sin_ref[...]
                              + jnp.dot(pool_sc[...], Wo_ref[...],
                                        preferred_element_type=jnp.float32)
                              + bo_ref[...][None, :])


@jax.jit
def _tc_dense(h, agg2, W, b, g, be, batch, Wo, bo, score_in):
    zero = lambda p, i: (0, 0)
    first = lambda p, i: (i * (1 - p), 0)
    return pl.pallas_call(
        _tc_dense_body,
        grid=(2, NB),
        in_specs=[
            pl.BlockSpec((BR, D), first),
            pl.BlockSpec((NC, BR, D), lambda p, i: (0, i * (1 - p), 0)),
            pl.BlockSpec((D, D), zero),
            pl.BlockSpec((D,), lambda p, i: (0,)),
            pl.BlockSpec((D,), lambda p, i: (0,)),
            pl.BlockSpec((D,), lambda p, i: (0,)),
            pl.BlockSpec((1, 1, BR), lambda p, i: (p * i, 0, 0)),
            pl.BlockSpec((D, OUT_DIM), zero),
            pl.BlockSpec((OUT_DIM,), lambda p, i: (0,)),
            pl.BlockSpec((NG, OUT_DIM), zero),
        ],
        out_specs=(pl.BlockSpec((BR, D), lambda p, i: (p * i, 0)),
                   pl.BlockSpec((NG, OUT_DIM), zero)),
        out_shape=(jax.ShapeDtypeStruct((N, D), jnp.float32),
                   jax.ShapeDtypeStruct((NG, OUT_DIM), jnp.float32)),
        scratch_shapes=[
            pltpu.VMEM((N, D), jnp.float32),
            pltpu.VMEM((8, D), jnp.float32),
            pltpu.VMEM((8, D), jnp.float32),
            pltpu.VMEM((NG, D), jnp.float32),
        ],
    )(h, agg2.reshape(NC, N, D), W, b, g, be,
      batch.reshape(NB, 1, BR), Wo, bo, score_in)


def kernel(x, edge_index, batch, W0, b0, g0, be0, W1, b1, g1, be1,
           W2, b2, g2, be2, Wo0, bo0, Wo1, bo1, Wo2, bo2):
    ei = edge_index.reshape(2 * E)
    score = jnp.zeros((NG, OUT_DIM), jnp.float32)

    agg = _sc_agg(x, ei)
    h0, score = _tc_dense(x, agg, W0, b0, g0, be0, batch, Wo0, bo0, score)
    agg = _sc_agg(h0, ei)
    h1, score = _tc_dense(h0, agg, W1, b1, g1, be1, batch, Wo1, bo1, score)
    agg = _sc_agg(h1, ei)
    h2, score = _tc_dense(h1, agg, W2, b2, g2, be2, batch, Wo2, bo2, score)

    return score
